# baseline (speedup 1.0000x reference)
import functools

import jax
import jax.numpy as jnp
import numpy as np
from jax import lax
from jax.experimental import pallas as pl
from jax.experimental.pallas import tpu as pltpu
from jax.experimental.pallas import tpu_sc as plsc

_VOCAB = 100000
_D = 128
_DW = _D // 2
_H1 = 512
_H2 = 256
_NCLS = 11
_B = 4096
_L = 200
_EPS = 1e-5

_NCORES = 2
_NSUB = 16
_NW = _NCORES * _NSUB
_BPW = _B // _NW
_LH = 104
_LRAW = _L // 2

_PERM = np.concatenate(
    [np.concatenate([g * 32 + np.arange(0, 32, 2), g * 32 + np.arange(1, 32, 2)])
     for g in range(_D // 32)])


def _embed_pool(x_flat, emb_i):
    mesh = plsc.VectorSubcoreMesh(core_axis_name="c", subcore_axis_name="s")

    @functools.partial(
        pl.kernel,
        out_type=jax.ShapeDtypeStruct((_B * _D,), jnp.float32),
        mesh=mesh,
        scratch_types=[
            pltpu.VMEM((_BPW * 2 * _LH,), jnp.int32),
            pltpu.VMEM((2, 2, _LH, _DW), jnp.int32),
            pltpu.VMEM((_BPW * _D,), jnp.float32),
            pltpu.SemaphoreType.DMA,
            pltpu.SemaphoreType.DMA,
        ],
        compiler_params=pltpu.CompilerParams(use_tc_tiling_on_sc=False),
    )
    def k(x_hbm, emb_hbm, out_hbm, idx_v, rows_v, acc_v, sem0, sem1):
        wid = lax.axis_index("s") * _NCORES + lax.axis_index("c")
        base = wid * _BPW
        emb2 = emb_hbm
        pltpu.sync_copy(x_hbm.at[pl.ds(base * 2 * _LH, _BPW * 2 * _LH)], idx_v)
        sems = (sem0, sem1)

        def issue(r, buf):
            for j in range(2):
                pltpu.async_copy(
                    emb2.at[idx_v.at[pl.ds(r * 2 * _LH + j * _LH, _LH)]],
                    rows_v.at[buf, j], sems[buf])

        def wait_buf(buf):
            for j in range(2):
                pltpu.make_async_copy(
                    emb2.at[idx_v.at[pl.ds(j * _LH, _LH)]],
                    rows_v.at[buf, j], sems[buf]).wait()

        hi_mask = jnp.full((16,), -65536, jnp.int32)

        def reduce_row(r, buf):
            def l_body(l, acc):
                a = list(acc)
                for j in range(2):
                    for g in range(4):
                        v = rows_v[buf, j, l, pl.ds(g * 16, 16)]
                        lo = lax.bitcast_convert_type(v << 16, jnp.float32)
                        hi = lax.bitcast_convert_type(v & hi_mask, jnp.float32)
                        a[2 * g] = a[2 * g] + lo
                        a[2 * g + 1] = a[2 * g + 1] + hi
                return tuple(a)

            acc = lax.fori_loop(
                0, _LH, l_body,
                tuple(jnp.zeros((16,), jnp.float32) for _ in range(8)),
                unroll=4)
            for d in range(8):
                acc_v[pl.ds(r * _D + d * 16, 16)] = acc[d] * (1.0 / _L)

        issue(0, 0)

        def pair_body(p, _):
            r0 = 2 * p
            issue(r0 + 1, 1)
            wait_buf(0)
            reduce_row(r0, 0)

            @pl.when(r0 + 2 < _BPW)
            def _():
                issue(r0 + 2, 0)

            wait_buf(1)
            reduce_row(r0 + 1, 1)
            return 0

        lax.fori_loop(0, _BPW // 2, pair_body, 0)
        pltpu.sync_copy(acc_v, out_hbm.at[pl.ds(base * _D, _BPW * _D)])

    return k(x_flat, emb_i)


def _mlp(pooled, W1, v1, W2, v2, W3, b3):
    BM = 512
    grid = (_B // BM,)

    def body(p_ref, W1_ref, v1_ref, W2_ref, v2_ref, W3_ref, b3_ref, o_ref):
        p = p_ref[:]
        h = jnp.dot(p, W1_ref[:], preferred_element_type=jnp.float32)
        b, g, be, rm, rv = (v1_ref[i:i + 1, :] for i in range(5))
        s = g * lax.rsqrt(rv + _EPS)
        h = jnp.maximum(h * s + (b - rm) * s + be, 0.0)
        h = jnp.dot(h, W2_ref[:], preferred_element_type=jnp.float32)
        b, g, be, rm, rv = (v2_ref[i:i + 1, :] for i in range(5))
        s = g * lax.rsqrt(rv + _EPS)
        h = jnp.maximum(h * s + (b - rm) * s + be, 0.0)
        o_ref[:] = (jnp.dot(h, W3_ref[:], preferred_element_type=jnp.float32)
                    + b3_ref[:])

    rep = lambda shape: pl.BlockSpec(shape, lambda i: (0,) * len(shape))
    return pl.pallas_call(
        body,
        grid=grid,
        in_specs=[
            pl.BlockSpec((BM, _D), lambda i: (i, 0)),
            rep((_D, _H1)), rep((5, _H1)),
            rep((_H1, _H2)), rep((5, _H2)),
            rep((_H2, _NCLS)), rep((1, _NCLS)),
        ],
        out_specs=pl.BlockSpec((BM, _NCLS), lambda i: (i, 0)),
        out_shape=jax.ShapeDtypeStruct((_B, _NCLS), jnp.float32),
    )(pooled, W1, v1, W2, v2, W3, b3)


def kernel(x, emb, W1, b1, g1, be1, rm1, rv1, W2, b2, g2, be2, rm2, rv2, W3, b3):
    x3 = x.astype(jnp.int32).reshape(_B, 2, _LRAW)
    x3 = jnp.pad(x3, ((0, 0), (0, 0), (0, _LH - _LRAW)))
    emb_i = lax.bitcast_convert_type(
        emb.astype(jnp.bfloat16).reshape(_VOCAB, _DW, 2), jnp.int32)
    pooled = _embed_pool(x3.reshape(-1), emb_i).reshape(_B, _D)
    v1 = jnp.stack([b1, g1, be1, rm1, rv1])
    v2 = jnp.stack([b2, g2, be2, rm2, rv2])
    W1p = W1[_PERM, :]
    return _mlp(pooled, W1p, v1, W2, v2, W3, b3.reshape(1, _NCLS))


# speedup vs baseline: 5.3499x; 5.3499x over previous
import functools

import jax
import jax.numpy as jnp
from jax import lax
from jax.experimental import pallas as pl
from jax.experimental.pallas import tpu as pltpu
from jax.experimental.pallas import tpu_sc as plsc

_VOCAB = 100000
_D = 128
_H1 = 512
_H2 = 256
_NCLS = 11
_B = 4096
_L = 200
_EPS = 1e-5

_NCORES = 2
_NSUB = 16
_NW = _NCORES * _NSUB
_BPW = _B // _NW
_LH = _L // 2


def _embed_pool(x3, emb):
    mesh = plsc.VectorSubcoreMesh(core_axis_name="c", subcore_axis_name="s")

    @functools.partial(
        pl.kernel,
        out_type=jax.ShapeDtypeStruct((_B, _D), jnp.float32),
        mesh=mesh,
        scratch_types=[
            pltpu.VMEM((_BPW, 2, _LH), jnp.int32),
            pltpu.VMEM((2, 2, _LH, _D), jnp.float32),
            pltpu.VMEM((_BPW, _D), jnp.float32),
            pltpu.SemaphoreType.DMA,
            pltpu.SemaphoreType.DMA,
        ],
    )
    def k(x_hbm, emb_hbm, out_hbm, idx_v, rows_v, acc_v, sem0, sem1):
        wid = lax.axis_index("s") * _NCORES + lax.axis_index("c")
        base = wid * _BPW
        pltpu.sync_copy(x_hbm.at[pl.ds(base, _BPW)], idx_v)
        sems = (sem0, sem1)

        def issue(r, buf):
            for j in range(2):
                pltpu.async_copy(
                    emb_hbm.at[idx_v.at[r, j]], rows_v.at[buf, j], sems[buf])

        def wait_buf(buf):
            for j in range(2):
                pltpu.make_async_copy(
                    emb_hbm.at[idx_v.at[0, j]], rows_v.at[buf, j],
                    sems[buf]).wait()

        def reduce_row(r, buf):
            def l_body(l, acc):
                a = list(acc)
                for j in range(2):
                    for d in range(8):
                        a[d] = a[d] + rows_v[buf, j, l, pl.ds(d * 16, 16)]
                return tuple(a)

            acc = lax.fori_loop(
                0, _LH, l_body,
                tuple(jnp.zeros((16,), jnp.float32) for _ in range(8)),
                unroll=10)
            for d in range(8):
                acc_v[r, pl.ds(d * 16, 16)] = acc[d] * (1.0 / _L)

        issue(0, 0)

        def pair_body(p, _):
            r0 = 2 * p
            issue(r0 + 1, 1)
            wait_buf(0)
            reduce_row(r0, 0)

            @pl.when(r0 + 2 < _BPW)
            def _():
                issue(r0 + 2, 0)

            wait_buf(1)
            reduce_row(r0 + 1, 1)
            return 0

        lax.fori_loop(0, _BPW // 2, pair_body, 0)
        pltpu.sync_copy(acc_v, out_hbm.at[pl.ds(base, _BPW)])

    return k(x3, emb)


def _mlp(pooled, W1, v1, W2, v2, W3, b3):
    BM = 512
    grid = (_B // BM,)

    def body(p_ref, W1_ref, v1_ref, W2_ref, v2_ref, W3_ref, b3_ref, o_ref):
        p = p_ref[:]
        h = jnp.dot(p, W1_ref[:], preferred_element_type=jnp.float32)
        b, g, be, rm, rv = (v1_ref[i:i + 1, :] for i in range(5))
        s = g * lax.rsqrt(rv + _EPS)
        h = jnp.maximum(h * s + (b - rm) * s + be, 0.0)
        h = jnp.dot(h, W2_ref[:], preferred_element_type=jnp.float32)
        b, g, be, rm, rv = (v2_ref[i:i + 1, :] for i in range(5))
        s = g * lax.rsqrt(rv + _EPS)
        h = jnp.maximum(h * s + (b - rm) * s + be, 0.0)
        o_ref[:] = (jnp.dot(h, W3_ref[:], preferred_element_type=jnp.float32)
                    + b3_ref[:])

    rep = lambda shape: pl.BlockSpec(shape, lambda i: (0,) * len(shape))
    return pl.pallas_call(
        body,
        grid=grid,
        in_specs=[
            pl.BlockSpec((BM, _D), lambda i: (i, 0)),
            rep((_D, _H1)), rep((5, _H1)),
            rep((_H1, _H2)), rep((5, _H2)),
            rep((_H2, _NCLS)), rep((1, _NCLS)),
        ],
        out_specs=pl.BlockSpec((BM, _NCLS), lambda i: (i, 0)),
        out_shape=jax.ShapeDtypeStruct((_B, _NCLS), jnp.float32),
    )(pooled, W1, v1, W2, v2, W3, b3)


def kernel(x, emb, W1, b1, g1, be1, rm1, rv1, W2, b2, g2, be2, rm2, rv2, W3, b3):
    x3 = x.astype(jnp.int32).reshape(_B, 2, _LH)
    pooled = _embed_pool(x3, emb)
    v1 = jnp.stack([b1, g1, be1, rm1, rv1])
    v2 = jnp.stack([b2, g2, be2, rm2, rv2])
    return _mlp(pooled, W1, v1, W2, v2, W3, b3.reshape(1, _NCLS))


# speedup vs baseline: 5.8339x; 1.0905x over previous
import functools

import jax
import jax.numpy as jnp
from jax import lax
from jax.experimental import pallas as pl
from jax.experimental.pallas import tpu as pltpu
from jax.experimental.pallas import tpu_sc as plsc

_VOCAB = 100000
_D = 128
_H1 = 512
_H2 = 256
_NCLS = 11
_B = 4096
_L = 200
_EPS = 1e-5

_NCORES = 2
_NSUB = 16
_NW = _NCORES * _NSUB
_BPW = _B // _NW
_LH = _L // 2


def _embed_pool(x3, emb):
    mesh = plsc.VectorSubcoreMesh(core_axis_name="c", subcore_axis_name="s")

    @functools.partial(
        pl.kernel,
        out_type=jax.ShapeDtypeStruct((_B, _D), jnp.float32),
        mesh=mesh,
        scratch_types=[
            pltpu.VMEM((_BPW, 2, _LH), jnp.int32),
            pltpu.VMEM((2, 2, _LH, _D), jnp.float32),
            pltpu.VMEM((_BPW, _D), jnp.float32),
            pltpu.SemaphoreType.DMA,
            pltpu.SemaphoreType.DMA,
        ],
    )
    def k(x_hbm, emb_hbm, out_hbm, idx_v, rows_v, acc_v, sem0, sem1):
        wid = lax.axis_index("s") * _NCORES + lax.axis_index("c")
        base = wid * _BPW
        pltpu.sync_copy(x_hbm.at[pl.ds(base, _BPW)], idx_v)
        sems = (sem0, sem1)

        def issue(r, buf):
            for j in range(2):
                pltpu.async_copy(
                    emb_hbm.at[idx_v.at[r, j]], rows_v.at[buf, j], sems[buf])

        def wait_buf(buf):
            for j in range(2):
                pltpu.make_async_copy(
                    emb_hbm.at[idx_v.at[0, j]], rows_v.at[buf, j],
                    sems[buf]).wait()

        def reduce_row(r, buf):
            def l_body(l, acc):
                a = list(acc)
                for j in range(2):
                    for d in range(8):
                        a[d] = a[d] + rows_v[buf, j, l, pl.ds(d * 16, 16)]
                return tuple(a)

            acc = lax.fori_loop(
                0, _LH, l_body,
                tuple(jnp.zeros((16,), jnp.float32) for _ in range(8)),
                unroll=4)
            for d in range(8):
                acc_v[r, pl.ds(d * 16, 16)] = acc[d] * (1.0 / _L)

        issue(0, 0)

        def pair_body(p, _):
            r0 = 2 * p
            issue(r0 + 1, 1)
            wait_buf(0)
            reduce_row(r0, 0)

            @pl.when(r0 + 2 < _BPW)
            def _():
                issue(r0 + 2, 0)

            wait_buf(1)
            reduce_row(r0 + 1, 1)
            return 0

        lax.fori_loop(0, _BPW // 2, pair_body, 0)
        pltpu.sync_copy(acc_v, out_hbm.at[pl.ds(base, _BPW)])

    return k(x3, emb)


def _mlp(pooled, W1, v1, W2, v2, W3, b3):
    BM = 512
    grid = (_B // BM,)

    def body(p_ref, W1_ref, v1_ref, W2_ref, v2_ref, W3_ref, b3_ref, o_ref):
        p = p_ref[:]
        h = jnp.dot(p, W1_ref[:], preferred_element_type=jnp.float32)
        b, g, be, rm, rv = (v1_ref[i:i + 1, :] for i in range(5))
        s = g * lax.rsqrt(rv + _EPS)
        h = jnp.maximum(h * s + (b - rm) * s + be, 0.0)
        h = jnp.dot(h, W2_ref[:], preferred_element_type=jnp.float32)
        b, g, be, rm, rv = (v2_ref[i:i + 1, :] for i in range(5))
        s = g * lax.rsqrt(rv + _EPS)
        h = jnp.maximum(h * s + (b - rm) * s + be, 0.0)
        o_ref[:] = (jnp.dot(h, W3_ref[:], preferred_element_type=jnp.float32)
                    + b3_ref[:])

    rep = lambda shape: pl.BlockSpec(shape, lambda i: (0,) * len(shape))
    return pl.pallas_call(
        body,
        grid=grid,
        in_specs=[
            pl.BlockSpec((BM, _D), lambda i: (i, 0)),
            rep((_D, _H1)), rep((5, _H1)),
            rep((_H1, _H2)), rep((5, _H2)),
            rep((_H2, _NCLS)), rep((1, _NCLS)),
        ],
        out_specs=pl.BlockSpec((BM, _NCLS), lambda i: (i, 0)),
        out_shape=jax.ShapeDtypeStruct((_B, _NCLS), jnp.float32),
    )(pooled, W1, v1, W2, v2, W3, b3)


def kernel(x, emb, W1, b1, g1, be1, rm1, rv1, W2, b2, g2, be2, rm2, rv2, W3, b3):
    x3 = x.astype(jnp.int32).reshape(_B, 2, _LH)
    pooled = _embed_pool(x3, emb)
    v1 = jnp.stack([b1, g1, be1, rm1, rv1])
    v2 = jnp.stack([b2, g2, be2, rm2, rv2])
    return _mlp(pooled, W1, v1, W2, v2, W3, b3.reshape(1, _NCLS))


# speedup vs baseline: 7.0362x; 1.2061x over previous
import functools

import jax
import jax.numpy as jnp
from jax import lax
from jax.experimental import pallas as pl
from jax.experimental.pallas import tpu as pltpu
from jax.experimental.pallas import tpu_sc as plsc

_VOCAB = 100000
_D = 128
_H1 = 512
_H2 = 256
_NCLS = 11
_B = 4096
_L = 200
_EPS = 1e-5

_NCORES = 2
_NSUB = 16
_NW = _NCORES * _NSUB
_BPW = _B // _NW
_LQ = _L // 4
_NR = 8


def _embed_pool(x4, emb):
    mesh = plsc.VectorSubcoreMesh(core_axis_name="c", subcore_axis_name="s")

    @functools.partial(
        pl.kernel,
        out_type=jax.ShapeDtypeStruct((_B, _D), jnp.float32),
        mesh=mesh,
        scratch_types=[
            pltpu.VMEM((_BPW, 4, _LQ), jnp.int32),
            pltpu.VMEM((_NR, _LQ, _D), jnp.float32),
            pltpu.VMEM((32, _D), jnp.float32),
        ] + [pltpu.SemaphoreType.DMA] * _NR,
    )
    def k(x_hbm, emb_hbm, out_hbm, idx_v, rows_v, acc_v, *sems):
        wid = lax.axis_index("s") * _NCORES + lax.axis_index("c")
        base = wid * _BPW
        pltpu.sync_copy(x_hbm.at[pl.ds(base, _BPW)], idx_v)

        def issue(r, c, slot):
            pltpu.async_copy(
                emb_hbm.at[idx_v.at[r, c]], rows_v.at[slot], sems[slot])

        def wait_slot(slot):
            pltpu.make_async_copy(
                emb_hbm.at[idx_v.at[0, 0]], rows_v.at[slot],
                sems[slot]).wait()

        def reduce_quarter(slot, acc):
            def l_body(l, acc):
                return tuple(acc[d] + rows_v[slot, l, pl.ds(d * 16, 16)]
                             for d in range(8))

            return lax.fori_loop(0, _LQ, l_body, acc, unroll=5)

        for k8 in range(_NR):
            issue(k8 // 4, k8 % 4, k8)

        zacc = tuple(jnp.zeros((16,), jnp.float32) for _ in range(8))

        def step_body(s, _):
            for half in range(2):
                r = 2 * s + half
                acc = zacc
                for c in range(4):
                    slot = 4 * half + c
                    wait_slot(slot)
                    acc = reduce_quarter(slot, acc)

                    @pl.when(s < _BPW // 2 - 1)
                    def _():
                        issue(r + 2, c, slot)

                ra = jnp.bitwise_and(r, 31)
                for d in range(8):
                    acc_v[ra, pl.ds(d * 16, 16)] = acc[d] * (1.0 / _L)

            @pl.when(jnp.bitwise_and(s, 15) == 15)
            def _():
                pltpu.sync_copy(
                    acc_v, out_hbm.at[pl.ds(pl.multiple_of(base + 2 * s - 30, 32), 32)])

            return 0

        lax.fori_loop(0, _BPW // 2, step_body, 0)

    return k(x4, emb)


def _mlp(pooled, W1, v1, W2, v2, W3, b3):
    BM = 512
    grid = (_B // BM,)

    def body(p_ref, W1_ref, v1_ref, W2_ref, v2_ref, W3_ref, b3_ref, o_ref):
        p = p_ref[:]
        h = jnp.dot(p, W1_ref[:], preferred_element_type=jnp.float32)
        b, g, be, rm, rv = (v1_ref[i:i + 1, :] for i in range(5))
        s = g * lax.rsqrt(rv + _EPS)
        h = jnp.maximum(h * s + (b - rm) * s + be, 0.0)
        h = jnp.dot(h, W2_ref[:], preferred_element_type=jnp.float32)
        b, g, be, rm, rv = (v2_ref[i:i + 1, :] for i in range(5))
        s = g * lax.rsqrt(rv + _EPS)
        h = jnp.maximum(h * s + (b - rm) * s + be, 0.0)
        o_ref[:] = (jnp.dot(h, W3_ref[:], preferred_element_type=jnp.float32)
                    + b3_ref[:])

    rep = lambda shape: pl.BlockSpec(shape, lambda i: (0,) * len(shape))
    return pl.pallas_call(
        body,
        grid=grid,
        in_specs=[
            pl.BlockSpec((BM, _D), lambda i: (i, 0)),
            rep((_D, _H1)), rep((5, _H1)),
            rep((_H1, _H2)), rep((5, _H2)),
            rep((_H2, _NCLS)), rep((1, _NCLS)),
        ],
        out_specs=pl.BlockSpec((BM, _NCLS), lambda i: (i, 0)),
        out_shape=jax.ShapeDtypeStruct((_B, _NCLS), jnp.float32),
    )(pooled, W1, v1, W2, v2, W3, b3)


def kernel(x, emb, W1, b1, g1, be1, rm1, rv1, W2, b2, g2, be2, rm2, rv2, W3, b3):
    x4 = x.astype(jnp.int32).reshape(_B, 4, _LQ)
    pooled = _embed_pool(x4, emb)
    v1 = jnp.stack([b1, g1, be1, rm1, rv1])
    v2 = jnp.stack([b2, g2, be2, rm2, rv2])
    return _mlp(pooled, W1, v1, W2, v2, W3, b3.reshape(1, _NCLS))


# speedup vs baseline: 7.1323x; 1.0137x over previous
import functools

import jax
import jax.numpy as jnp
from jax import lax
from jax.experimental import pallas as pl
from jax.experimental.pallas import tpu as pltpu
from jax.experimental.pallas import tpu_sc as plsc

_VOCAB = 100000
_D = 128
_H1 = 512
_H2 = 256
_NCLS = 11
_B = 4096
_L = 200
_EPS = 1e-5

_NCORES = 2
_NSUB = 16
_NW = _NCORES * _NSUB
_BPW = _B // _NW
_LQ = _L // 4
_NR = 8


def _embed_pool(x4, emb):
    mesh = plsc.VectorSubcoreMesh(core_axis_name="c", subcore_axis_name="s")

    @functools.partial(
        pl.kernel,
        out_type=jax.ShapeDtypeStruct((_B, _D), jnp.float32),
        mesh=mesh,
        scratch_types=[
            pltpu.VMEM((_BPW, 4, _LQ), jnp.int32),
            pltpu.VMEM((_NR, _LQ, _D), jnp.float32),
            pltpu.VMEM((32, _D), jnp.float32),
        ] + [pltpu.SemaphoreType.DMA] * _NR,
    )
    def k(x_hbm, emb_hbm, out_hbm, idx_v, rows_v, acc_v, *sems):
        wid = lax.axis_index("s") * _NCORES + lax.axis_index("c")
        base = wid * _BPW
        pltpu.sync_copy(x_hbm.at[pl.ds(base, _BPW)], idx_v)

        def issue(r, c, slot):
            pltpu.async_copy(
                emb_hbm.at[idx_v.at[r, c]], rows_v.at[slot], sems[slot])

        def wait_slot(slot):
            pltpu.make_async_copy(
                emb_hbm.at[idx_v.at[0, 0]], rows_v.at[slot],
                sems[slot]).wait()

        def reduce_quarter(slot, acc):
            def l_body(l, acc):
                return tuple(acc[d] + rows_v[slot, l, pl.ds(d * 16, 16)]
                             for d in range(8))

            return lax.fori_loop(0, _LQ, l_body, acc, unroll=10)

        for k8 in range(_NR):
            issue(k8 // 4, k8 % 4, k8)

        zacc = tuple(jnp.zeros((16,), jnp.float32) for _ in range(8))

        def step_body(s, _):
            for half in range(2):
                r = 2 * s + half
                acc = zacc
                for c in range(4):
                    slot = 4 * half + c
                    wait_slot(slot)
                    acc = reduce_quarter(slot, acc)

                    @pl.when(s < _BPW // 2 - 1)
                    def _():
                        issue(r + 2, c, slot)

                ra = jnp.bitwise_and(r, 31)
                for d in range(8):
                    acc_v[ra, pl.ds(d * 16, 16)] = acc[d] * (1.0 / _L)

            @pl.when(jnp.bitwise_and(s, 15) == 15)
            def _():
                pltpu.sync_copy(
                    acc_v, out_hbm.at[pl.ds(pl.multiple_of(base + 2 * s - 30, 32), 32)])

            return 0

        lax.fori_loop(0, _BPW // 2, step_body, 0)

    return k(x4, emb)


def _mlp(pooled, W1, v1, W2, v2, W3, b3):
    BM = 2048
    grid = (_B // BM,)

    def body(p_ref, W1_ref, v1_ref, W2_ref, v2_ref, W3_ref, b3_ref, o_ref):
        p = p_ref[:]
        h = jnp.dot(p, W1_ref[:], preferred_element_type=jnp.float32)
        b, g, be, rm, rv = (v1_ref[i:i + 1, :] for i in range(5))
        s = g * lax.rsqrt(rv + _EPS)
        h = jnp.maximum(h * s + (b - rm) * s + be, 0.0)
        h = jnp.dot(h, W2_ref[:], preferred_element_type=jnp.float32)
        b, g, be, rm, rv = (v2_ref[i:i + 1, :] for i in range(5))
        s = g * lax.rsqrt(rv + _EPS)
        h = jnp.maximum(h * s + (b - rm) * s + be, 0.0)
        o_ref[:] = (jnp.dot(h, W3_ref[:], preferred_element_type=jnp.float32)
                    + b3_ref[:])

    rep = lambda shape: pl.BlockSpec(shape, lambda i: (0,) * len(shape))
    return pl.pallas_call(
        body,
        grid=grid,
        in_specs=[
            pl.BlockSpec((BM, _D), lambda i: (i, 0)),
            rep((_D, _H1)), rep((5, _H1)),
            rep((_H1, _H2)), rep((5, _H2)),
            rep((_H2, _NCLS)), rep((1, _NCLS)),
        ],
        out_specs=pl.BlockSpec((BM, _NCLS), lambda i: (i, 0)),
        out_shape=jax.ShapeDtypeStruct((_B, _NCLS), jnp.float32),
    )(pooled, W1, v1, W2, v2, W3, b3)


def kernel(x, emb, W1, b1, g1, be1, rm1, rv1, W2, b2, g2, be2, rm2, rv2, W3, b3):
    x4 = x.astype(jnp.int32).reshape(_B, 4, _LQ)
    pooled = _embed_pool(x4, emb)
    v1 = jnp.stack([b1, g1, be1, rm1, rv1])
    v2 = jnp.stack([b2, g2, be2, rm2, rv2])
    return _mlp(pooled, W1, v1, W2, v2, W3, b3.reshape(1, _NCLS))
